# baseline (speedup 1.0000x reference)
import functools

import jax
import jax.numpy as jnp
from jax import lax
from jax.experimental import pallas as pl
from jax.experimental.pallas import tpu as pltpu
from jax.experimental.pallas import tpu_sc as plsc

_LANES = 16


def _make_packer(n_rows, dim, rows_per_chunk, num_workers, num_cores):
    rows_per_worker = n_rows // num_workers
    chunks = rows_per_worker // rows_per_chunk
    chunk_elems = rows_per_chunk * dim
    groups = chunk_elems // _LANES

    mesh = plsc.VectorSubcoreMesh(core_axis_name="c", subcore_axis_name="s")

    @functools.partial(
        pl.kernel,
        mesh=mesh,
        out_type=jax.ShapeDtypeStruct((n_rows * dim,), jnp.float32),
        scratch_types=[
            pltpu.VMEM((chunk_elems,), jnp.int32),
            pltpu.VMEM((chunk_elems + _LANES,), jnp.float32),
            pltpu.VMEM((chunk_elems,), jnp.float32),
        ],
        compiler_params=pltpu.CompilerParams(needs_layout_passes=False),
    )
    def packer(x_hbm, idx_hbm, out_hbm, idx_v, buf, obuf):
        wid = lax.axis_index("s") * num_cores + lax.axis_index("c")
        base = wid * rows_per_worker * dim
        pltpu.sync_copy(idx_hbm, idx_v)
        buf[pl.ds(chunk_elems, _LANES)] = jnp.zeros((_LANES,), jnp.float32)

        def chunk_body(i, carry):
            off = base + i * chunk_elems
            pltpu.sync_copy(x_hbm.at[pl.ds(off, chunk_elems)],
                            buf.at[pl.ds(0, chunk_elems)])

            def grp(g, c):
                idx = idx_v[pl.ds(g * _LANES, _LANES)]
                obuf[pl.ds(g * _LANES, _LANES)] = plsc.load_gather(buf, [idx])
                return c

            lax.fori_loop(0, groups, grp, 0)
            pltpu.sync_copy(obuf, out_hbm.at[pl.ds(off, chunk_elems)])
            return carry

        lax.fori_loop(0, chunks, chunk_body, 0)

    return packer


def kernel(x_flat, pack_index, pad_mask):
    n, dim = x_flat.shape
    num_coeffs, num_channels = pad_mask.shape[1], pad_mask.shape[2]
    info = plsc.get_sparse_core_info()
    num_workers = info.num_cores * info.num_subcores
    rows_per_chunk = 25
    assert n % (num_workers * rows_per_chunk) == 0

    chunk_elems = rows_per_chunk * dim
    pidx = pack_index.astype(jnp.int32)
    mask_flat = pad_mask.reshape(-1)
    offs = jnp.arange(rows_per_chunk, dtype=jnp.int32)[:, None] * dim
    idx_full = jnp.where(mask_flat[None, :], chunk_elems,
                         pidx[None, :] + offs).reshape(-1)

    packer = _make_packer(n, dim, rows_per_chunk, num_workers, info.num_cores)
    out_flat = packer(x_flat.reshape(-1), idx_full)
    return out_flat.reshape(n, num_coeffs, num_channels)


# speedup vs baseline: 1.7739x; 1.7739x over previous
import functools

import jax
import jax.numpy as jnp
from jax import lax
from jax.experimental import pallas as pl
from jax.experimental.pallas import tpu as pltpu
from jax.experimental.pallas import tpu_sc as plsc

_LANES = 16
_ROWS_PER_CHUNK = 20
_UNROLL = 8


def _make_packer(n_rows, dim, num_workers, num_cores):
    rows = _ROWS_PER_CHUNK
    rows_per_worker = n_rows // num_workers
    chunks = (rows_per_worker + rows - 1) // rows
    chunk_elems = rows * dim
    main_chunks = (chunks - 1) if (chunks % 2) else chunks

    mesh = plsc.VectorSubcoreMesh(core_axis_name="c", subcore_axis_name="s")

    @functools.partial(
        pl.kernel,
        mesh=mesh,
        out_type=jax.ShapeDtypeStruct((n_rows * dim,), jnp.float32),
        scratch_types=[
            pltpu.VMEM((chunk_elems,), jnp.int32),
            pltpu.VMEM((chunk_elems + _LANES,), jnp.float32),
            pltpu.VMEM((chunk_elems + _LANES,), jnp.float32),
            pltpu.VMEM((chunk_elems,), jnp.float32),
            pltpu.VMEM((chunk_elems,), jnp.float32),
            pltpu.SemaphoreType.DMA,
            pltpu.SemaphoreType.DMA,
            pltpu.SemaphoreType.DMA,
            pltpu.SemaphoreType.DMA,
        ],
        compiler_params=pltpu.CompilerParams(needs_layout_passes=False),
    )
    def packer(x_hbm, idx_hbm, out_hbm, idx_v, ib0, ib1, ob0, ob1,
               isem0, isem1, osem0, osem1):
        ibufs, obufs = (ib0, ib1), (ob0, ob1)
        isems, osems = (isem0, isem1), (osem0, osem1)
        wid = lax.axis_index("s") * num_cores + lax.axis_index("c")
        base = wid * rows_per_worker * dim
        last_off = (rows_per_worker - rows) * dim

        def chunk_off(ci):
            return base + jnp.minimum(ci * chunk_elems, last_off)

        def start_in(ci, b):
            pltpu.async_copy(x_hbm.at[pl.ds(chunk_off(ci), chunk_elems)],
                             ibufs[b].at[pl.ds(0, chunk_elems)], isems[b])

        def wait_in(b):
            pltpu.make_async_copy(x_hbm.at[pl.ds(0, chunk_elems)],
                                  ibufs[b].at[pl.ds(0, chunk_elems)],
                                  isems[b]).wait()

        def start_out(ci, b):
            pltpu.async_copy(obufs[b],
                             out_hbm.at[pl.ds(chunk_off(ci), chunk_elems)],
                             osems[b])

        def wait_out(b):
            pltpu.make_async_copy(obufs[b],
                                  out_hbm.at[pl.ds(0, chunk_elems)],
                                  osems[b]).wait()

        def compute(b):
            ib, ob = ibufs[b], obufs[b]

            @plsc.parallel_loop(0, chunk_elems, step=_LANES, unroll=_UNROLL)
            def _(s):
                idx = idx_v[pl.ds(s, _LANES)]
                ob[pl.ds(s, _LANES)] = plsc.load_gather(ib, [idx])

        pltpu.sync_copy(idx_hbm, idx_v)
        zeros = jnp.zeros((_LANES,), jnp.float32)
        ib0[pl.ds(chunk_elems, _LANES)] = zeros
        ib1[pl.ds(chunk_elems, _LANES)] = zeros
        start_in(0, 0)
        start_in(1, 1)

        @pl.loop(0, main_chunks, step=2)
        def _(i):
            for b in range(2):
                ci = i + b
                wait_in(b)

                @pl.when(ci >= 2)
                def _():
                    wait_out(b)

                compute(b)
                start_out(ci, b)

                @pl.when(ci + 2 < chunks)
                def _():
                    start_in(ci + 2, b)

        if main_chunks != chunks:
            wait_in(0)
            wait_out(0)
            compute(0)
            start_out(chunks - 1, 0)
        wait_out(0)
        wait_out(1)

    return packer


def kernel(x_flat, pack_index, pad_mask):
    n, dim = x_flat.shape
    num_coeffs, num_channels = pad_mask.shape[1], pad_mask.shape[2]
    info = plsc.get_sparse_core_info()
    num_workers = info.num_cores * info.num_subcores
    assert n % num_workers == 0 and n // num_workers >= _ROWS_PER_CHUNK

    chunk_elems = _ROWS_PER_CHUNK * dim
    pidx = pack_index.astype(jnp.int32)
    mask_flat = pad_mask.reshape(-1)
    offs = jnp.arange(_ROWS_PER_CHUNK, dtype=jnp.int32)[:, None] * dim
    idx_full = jnp.where(mask_flat[None, :], chunk_elems,
                         pidx[None, :] + offs).reshape(-1)

    packer = _make_packer(n, dim, num_workers, info.num_cores)
    out_flat = packer(x_flat.reshape(-1), idx_full)
    return out_flat.reshape(n, num_coeffs, num_channels)
